# baseline (speedup 1.0000x reference)
import functools

import jax
import jax.numpy as jnp
from jax import lax
from jax.experimental import pallas as pl
from jax.experimental.pallas import tpu as pltpu
from jax.experimental.pallas import tpu_sc as plsc

_CH = 128
_NTILES = 32
_NSUB = 16
_NG = 2


def _enc_body(x_ref, we_ref, be_ref, ws_ref, x0_ref, p_ref):
    x0 = jnp.dot(x_ref[...], we_ref[...],
                 preferred_element_type=jnp.float32) + be_ref[...]
    x0_ref[...] = x0
    p_ref[...] = jnp.dot(x0, ws_ref[...], preferred_element_type=jnp.float32)


def _mid_body(h_ref, x0_ref, o_ref, ws_ref, om_ref, beta_ref, h2_ref, p2_ref):
    h = h_ref[...]
    pre = o_ref[...] - h * om_ref[...] - beta_ref[0, 0] * x0_ref[...]
    h2 = h + jnp.maximum(pre, 0.0)
    h2_ref[...] = h2
    p2_ref[...] = jnp.dot(h2, ws_ref[...], preferred_element_type=jnp.float32)


def _dec_body(h_ref, x0_ref, o_ref, om_ref, beta_ref, wd_ref, bd_ref, y_ref):
    h = h_ref[...]
    pre = o_ref[...] - h * om_ref[...] - beta_ref[0, 0] * x0_ref[...]
    h3 = h + jnp.maximum(pre, 0.0)
    y_ref[...] = jnp.dot(h3, wd_ref[...],
                         preferred_element_type=jnp.float32) + bd_ref[...]


def _row_spec(blk, d):
    return pl.BlockSpec((blk, d), lambda i: (i, 0))


def _full_spec(shape):
    ndim = len(shape)
    return pl.BlockSpec(shape, lambda i: (0,) * ndim)


def _sc_aggregate(p_tab, src_t, dst_t, zeros_chunk):
    np2, dh = p_tab.shape
    np_ = np2 // 2
    _, k, ch = src_t.shape
    rpt = np_ // _NSUB

    mesh = plsc.VectorSubcoreMesh(core_axis_name="c", subcore_axis_name="s")

    @functools.partial(
        pl.kernel,
        mesh=mesh,
        out_type=jax.ShapeDtypeStruct((np_, 2, dh), jnp.float32),
        compiler_params=pltpu.CompilerParams(use_tc_tiling_on_sc=False),
        scratch_types=[
            pltpu.VMEM((k, ch), jnp.int32),
            pltpu.VMEM((k, ch), jnp.int32),
            *[pltpu.VMEM((ch, dh), jnp.float32) for _ in range(2 * _NG)],
            pltpu.VMEM_SHARED((np_, dh), jnp.float32),
            *[pltpu.SemaphoreType.DMA for _ in range(4)],
        ],
    )
    def agg_kernel(p_hbm, src_hbm, dst_hbm, zero_hbm, out_hbm,
                   src_v, dst_v, *rest):
        bufs_a = rest[:_NG]
        bufs_b = rest[_NG:2 * _NG]
        agg_sh = rest[2 * _NG]
        gsem_a, gsem_b, ssem_a, ssem_b = rest[2 * _NG + 1:]
        c = lax.axis_index("c")
        s = lax.axis_index("s")
        w = c * _NSUB + s
        row0 = s * rpt

        pltpu.sync_copy(zero_hbm, bufs_a[0])
        for r in range(rpt // ch):
            pltpu.sync_copy(bufs_a[0], agg_sh.at[pl.ds(row0 + r * ch, ch)])
        rem = rpt % ch
        if rem:
            pltpu.sync_copy(
                bufs_a[0].at[pl.ds(0, rem)],
                agg_sh.at[pl.ds(row0 + (rpt // ch) * ch, rem)])
        pltpu.sync_copy(src_hbm.at[w], src_v)
        pltpu.sync_copy(dst_hbm.at[w], dst_v)
        plsc.subcore_barrier()

        def fire_g(bufs, g, sem):
            for b in range(_NG):
                pltpu.async_copy(p_hbm.at[src_v.at[g * _NG + b]], bufs[b],
                                 sem)

        def wait_g(bufs, sem):
            for b in range(_NG):
                pltpu.make_async_copy(p_hbm.at[src_v.at[0]], bufs[b],
                                      sem).wait()

        def fire_s(bufs, g, sem):
            for b in range(_NG):
                pltpu.async_copy(bufs[b], agg_sh.at[dst_v.at[g * _NG + b]],
                                 sem, add=True)

        def wait_s(bufs, sem):
            for b in range(_NG):
                pltpu.make_async_copy(bufs[b], agg_sh.at[dst_v.at[0]],
                                      sem).wait()

        fire_g(bufs_a, 0, gsem_a)

        def body(u, carry):
            g0 = 2 * u
            g1 = 2 * u + 1
            fire_g(bufs_b, g1, gsem_b)
            wait_g(bufs_a, gsem_a)
            fire_s(bufs_a, g0, ssem_a)
            wait_g(bufs_b, gsem_b)
            fire_s(bufs_b, g1, ssem_b)
            wait_s(bufs_a, ssem_a)
            fire_g(bufs_a, g0 + 2, gsem_a)
            wait_s(bufs_b, ssem_b)
            return carry

        lax.fori_loop(0, k // (2 * _NG), body, 0)
        wait_g(bufs_a, gsem_a)
        plsc.subcore_barrier()

        pltpu.sync_copy(agg_sh.at[pl.ds(row0, rpt)],
                        out_hbm.at[pl.ds(row0, rpt), c])

    return agg_kernel(p_tab, src_t, dst_t, zeros_chunk)


def kernel(x, edge_index, W_enc, b_enc, W_dec, b_dec, W_pair, omega, beta):
    n, d = x.shape
    e = edge_index.shape[1]
    num_layers = 3

    rpt = -(-(n + 1) // (_NSUB * 8)) * 8
    np_ = _NSUB * rpt
    blk = np_ // 16
    dh = d // 2
    grp2 = 2 * _NG
    k = -(-e // (_NSUB * _CH * grp2)) * grp2
    k_idx = k + _NG
    ep = _NSUB * k_idx * _CH

    x_pad = jnp.zeros((np_, d), jnp.float32).at[:n].set(x)
    ws = 0.5 * (W_pair + W_pair.T)
    src = edge_index[0].astype(jnp.int32)
    dst = edge_index[1].astype(jnp.int32)
    npad = _NSUB * k * _CH - e
    pad_src = (jnp.arange(npad, dtype=jnp.int32) * 97) % np_
    pad_dst = n + jnp.arange(npad, dtype=jnp.int32) % (np_ - n)
    srcb = jnp.concatenate([src, pad_src]).reshape(_NSUB, k, _CH)
    dstb = jnp.concatenate([dst, pad_dst]).reshape(_NSUB, k, _CH)
    nov = _NSUB * _NG * _CH
    ov_src = ((jnp.arange(nov, dtype=jnp.int32) * 41) % np_).reshape(
        _NSUB, _NG, _CH)
    ov_dst = (n + jnp.arange(nov, dtype=jnp.int32) % (np_ - n)).reshape(
        _NSUB, _NG, _CH)
    srcb = jnp.concatenate([srcb, ov_src], axis=1)
    dstb = jnp.concatenate([dstb, ov_dst], axis=1)
    src_t = jnp.concatenate([2 * srcb, 2 * srcb + 1]).reshape(
        _NTILES, k_idx, _CH)
    dst_t = jnp.concatenate([dstb, dstb]).reshape(_NTILES, k_idx, _CH)
    zeros_chunk = jnp.zeros((_CH, dh), jnp.float32)
    b_enc2 = b_enc.reshape(1, d)
    b_dec2 = b_dec.reshape(1, d)
    om2 = omega.reshape(1, d)
    beta2 = jnp.reshape(beta, (1, 1)).astype(jnp.float32)

    grid = (np_ // blk,)
    row = _row_spec(blk, d)
    o_spec = row

    x0, p = pl.pallas_call(
        _enc_body,
        grid=grid,
        in_specs=[row, _full_spec((d, d)), _full_spec((1, d)),
                  _full_spec((d, d))],
        out_specs=[row, row],
        out_shape=[jax.ShapeDtypeStruct((np_, d), jnp.float32)] * 2,
    )(x_pad, W_enc, b_enc2, ws)

    h = x0
    y = None
    for layer in range(num_layers):
        parts = _sc_aggregate(p.reshape(2 * np_, dh), src_t, dst_t,
                              zeros_chunk)
        o2 = parts.reshape(np_, d)
        if layer < num_layers - 1:
            h, p = pl.pallas_call(
                _mid_body,
                grid=grid,
                in_specs=[row, row, o_spec, _full_spec((d, d)),
                          _full_spec((1, d)), _full_spec((1, 1))],
                out_specs=[row, row],
                out_shape=[jax.ShapeDtypeStruct((np_, d), jnp.float32)] * 2,
            )(h, x0, o2, ws, om2, beta2)
        else:
            y = pl.pallas_call(
                _dec_body,
                grid=grid,
                in_specs=[row, row, o_spec, _full_spec((1, d)),
                          _full_spec((1, 1)), _full_spec((d, d)),
                          _full_spec((1, d))],
                out_specs=row,
                out_shape=jax.ShapeDtypeStruct((np_, d), jnp.float32),
            )(h, x0, o2, om2, beta2, W_dec, b_dec2)

    return y[:n]
